# baseline (speedup 1.0000x reference)
import functools

import jax
import jax.numpy as jnp
from jax import lax
from jax.experimental import pallas as pl
from jax.experimental.pallas import tpu as pltpu
from jax.experimental.pallas import tpu_sc as plsc

IN_DIM = 2048
OUT_DIM = 2048
BATCH = 8192
HALF = BATCH // 2

_C0 = (0., 0., 0., 0., 0., 0., 0., 0., 1., 1., 1., 1., 1., 1., 1., 1.)
_C1 = (0., 0., 1., 1., 0., 0., 1., 1., -1., -1., 0., 0., -1., -1., 0., 0.)
_C2 = (0., 0., 0., 0., 1., 1., 1., 1., -1., -1., -1., -1., 0., 0., 0., 0.)
_C3 = (0., 1., -1., 0., -1., 0., -2., -1., 1., 2., 0., 1., 0., 1., -1., 0.)


def _first_argmax(w, iota, sentinel):
    mx = jnp.max(w, axis=1, keepdims=True)
    return jnp.min(jnp.where(w == mx, iota, sentinel), axis=1)


def _idx_coeff_body(lwa_ref, lwb_ref, maf_ref, mbf_ref, nw_ref, idx_ref, coef_ref):
    rows = lwa_ref.shape[0]
    iota_in = lax.broadcasted_iota(jnp.int32, (rows, IN_DIM), 1)
    wa = jnp.where(maf_ref[...] != 0, lwa_ref[...], -1e30)
    wb = jnp.where(mbf_ref[...] != 0, lwb_ref[...], -1e30)
    ia = _first_argmax(wa, iota_in, IN_DIM)
    ib = _first_argmax(wb, iota_in, IN_DIM)
    zeros_i = jnp.zeros((6, rows), jnp.int32)
    idx_ref[...] = jnp.concatenate([ia[None, :], ib[None, :], zeros_i], axis=0)

    iota_g = lax.broadcasted_iota(jnp.int32, (rows, 16), 1)
    g = _first_argmax(nw_ref[...], iota_g, 16)[None, :]
    c0 = jnp.zeros_like(g, dtype=jnp.float32)
    c1, c2, c3 = c0, c0, c0
    for k in range(16):
        sel = (g == k).astype(jnp.float32)
        c0 = c0 + sel * _C0[k]
        c1 = c1 + sel * _C1[k]
        c2 = c2 + sel * _C2[k]
        c3 = c3 + sel * _C3[k]
    zeros_f = jnp.zeros((4, rows), jnp.float32)
    coef_ref[...] = jnp.concatenate([c0, c1, c2, c3, zeros_f], axis=0)


def _bf16_bits(u):
    lsb = lax.shift_right_logical(u, 16) & 1
    return lax.shift_right_logical(u + 0x7FFF + lsb, 16)


def _pack_body(x0_ref, x1_ref, xt_ref):
    u0 = lax.bitcast_convert_type(x0_ref[...], jnp.int32)
    u1 = lax.bitcast_convert_type(x1_ref[...], jnp.int32)
    packed = lax.shift_left(_bf16_bits(u1), 16) | _bf16_bits(u0)
    xt_ref[...] = jnp.swapaxes(packed, 0, 1)


def _gate_body(a_ref, b_ref, coef_ref, out_ref):
    def unpack(u):
        lo = lax.bitcast_convert_type(lax.shift_left(u, 16), jnp.float32)
        hi = lax.bitcast_convert_type(u & jnp.int32(-65536), jnp.float32)
        return lo, hi

    half = out_ref.shape[0] // 2
    atp = jnp.swapaxes(a_ref[...], 0, 1)
    btp = jnp.swapaxes(b_ref[...], 0, 1)
    a_lo, a_hi = unpack(atp)
    b_lo, b_hi = unpack(btp)
    c0 = coef_ref[0, :][None, :]
    c1 = coef_ref[1, :][None, :]
    c2 = coef_ref[2, :][None, :]
    c3 = coef_ref[3, :][None, :]
    out_ref[pl.ds(0, half), :] = c0 + c1 * a_lo + c2 * b_lo + c3 * (a_lo * b_lo)
    out_ref[pl.ds(half, half), :] = c0 + c1 * a_hi + c2 * b_hi + c3 * (a_hi * b_hi)


def _sc_gather(xTp, idx_flat, width):
    mesh = plsc.VectorSubcoreMesh(core_axis_name="c", subcore_axis_name="s")
    chunk = 8
    n_chunks = 128 // chunk

    @functools.partial(
        pl.kernel,
        mesh=mesh,
        out_type=jax.ShapeDtypeStruct((2 * OUT_DIM, width), jnp.int32),
        scratch_types=[
            pltpu.VMEM((128,), jnp.int32),
            pltpu.VMEM((chunk, width), jnp.int32),
            pltpu.VMEM((chunk, width), jnp.int32),
            pltpu.SemaphoreType.DMA,
            pltpu.SemaphoreType.DMA,
        ],
    )
    def k(xT_hbm, idx_hbm, out_hbm, idx_v, buf0, buf1, gsem, wsem):
        wid = lax.axis_index("s") * 2 + lax.axis_index("c")
        base = wid * 128
        pltpu.sync_copy(idx_hbm.at[pl.ds(base, 128)], idx_v)
        bufs = (buf0, buf1)

        def start_gather(c):
            return pltpu.async_copy(
                xT_hbm.at[idx_v.at[pl.ds(c * chunk, chunk)]], bufs[c % 2], gsem
            )

        g = start_gather(0)
        wb = [None] * n_chunks
        for c in range(n_chunks):
            g.wait()
            if c + 1 < n_chunks:
                if c >= 1:
                    wb[c - 1].wait()
                g = start_gather(c + 1)
            wb[c] = pltpu.async_copy(
                bufs[c % 2], out_hbm.at[pl.ds(base + c * chunk, chunk)], wsem
            )
        wb[n_chunks - 2].wait()
        wb[n_chunks - 1].wait()

    return k(xTp, idx_flat)


def _stage1(lwa, lwb, maf, mbf, nw, interpret=False):
    blk = 256
    grid = OUT_DIM // blk
    return pl.pallas_call(
        _idx_coeff_body,
        grid=(grid,),
        in_specs=[
            pl.BlockSpec((blk, IN_DIM), lambda g: (g, 0)),
            pl.BlockSpec((blk, IN_DIM), lambda g: (g, 0)),
            pl.BlockSpec((blk, IN_DIM), lambda g: (g, 0)),
            pl.BlockSpec((blk, IN_DIM), lambda g: (g, 0)),
            pl.BlockSpec((blk, 16), lambda g: (g, 0)),
        ],
        out_specs=[
            pl.BlockSpec((8, blk), lambda g: (0, g)),
            pl.BlockSpec((8, blk), lambda g: (0, g)),
        ],
        out_shape=[
            jax.ShapeDtypeStruct((8, OUT_DIM), jnp.int32),
            jax.ShapeDtypeStruct((8, OUT_DIM), jnp.float32),
        ],
        compiler_params=pltpu.CompilerParams(
            dimension_semantics=("parallel",)),
        interpret=interpret,
    )(lwa, lwb, maf, mbf, nw)


QUARTER = BATCH // 4


def _stage2(x, h, interpret=False):
    rb, cb = 1024, 1024
    return pl.pallas_call(
        _pack_body,
        grid=(QUARTER // rb, IN_DIM // cb),
        in_specs=[
            pl.BlockSpec((rb, cb), lambda m, j: (4 * h + 2 * m, j)),
            pl.BlockSpec((rb, cb), lambda m, j: (4 * h + 2 * m + 1, j)),
        ],
        out_specs=pl.BlockSpec((cb, rb), lambda m, j: (j, m)),
        out_shape=jax.ShapeDtypeStruct((IN_DIM, QUARTER), jnp.int32),
        compiler_params=pltpu.CompilerParams(
            dimension_semantics=("parallel", "parallel")),
        interpret=interpret,
    )(x, x)


def _stage4(gT, coeffs, h, prev=None, interpret=False):
    nb, bb = 512, 1024
    gspecs = [
        pl.BlockSpec((nb, bb), lambda j, m: (j, m)),
        pl.BlockSpec((nb, bb), lambda j, m: (j + OUT_DIM // nb, m)),
        pl.BlockSpec((8, nb), lambda j, m: (0, j)),
    ]
    args = [gT, gT, coeffs]
    kwargs = {}
    body = _gate_body
    if prev is not None:
        gspecs.append(pl.BlockSpec((2 * bb, nb), lambda j, m: (0, 0)))
        args.append(prev)
        kwargs["input_output_aliases"] = {3: 0}

        def body(a_ref, b_ref, coef_ref, prev_ref, out_ref):
            del prev_ref
            _gate_body(a_ref, b_ref, coef_ref, out_ref)

    return pl.pallas_call(
        body,
        grid=(OUT_DIM // nb, QUARTER // bb),
        in_specs=gspecs,
        out_specs=pl.BlockSpec((2 * bb, nb), lambda j, m: (2 * h + m, j)),
        out_shape=jax.ShapeDtypeStruct((BATCH, OUT_DIM), jnp.float32),
        compiler_params=pltpu.CompilerParams(
            dimension_semantics=("parallel", "parallel")),
        interpret=interpret,
        **kwargs,
    )(*args)


def kernel(x, neuron_weights, link_weights_a, link_weights_b, link_mask_a, link_mask_b):
    ma8 = link_mask_a.astype(jnp.int8)
    mb8 = link_mask_b.astype(jnp.int8)
    idxmat, coeffs = _stage1(link_weights_a, link_weights_b, ma8,
                             mb8, neuron_weights)
    idx_flat = jnp.concatenate([idxmat[0], idxmat[1]])
    tA = _stage2(x, 0)
    gTA = _sc_gather(tA, idx_flat, QUARTER)
    tB = _stage2(x, 1)
    gTB = _sc_gather(tB, idx_flat, QUARTER)
    outA = _stage4(gTA, coeffs, 0)
    return _stage4(gTB, coeffs, 1, prev=outA)


# speedup vs baseline: 1.0174x; 1.0174x over previous
import functools

import jax
import jax.numpy as jnp
from jax import lax
from jax.experimental import pallas as pl
from jax.experimental.pallas import tpu as pltpu
from jax.experimental.pallas import tpu_sc as plsc

IN_DIM = 2048
OUT_DIM = 2048
BATCH = 8192
HALF = BATCH // 2

_C0 = (0., 0., 0., 0., 0., 0., 0., 0., 1., 1., 1., 1., 1., 1., 1., 1.)
_C1 = (0., 0., 1., 1., 0., 0., 1., 1., -1., -1., 0., 0., -1., -1., 0., 0.)
_C2 = (0., 0., 0., 0., 1., 1., 1., 1., -1., -1., -1., -1., 0., 0., 0., 0.)
_C3 = (0., 1., -1., 0., -1., 0., -2., -1., 1., 2., 0., 1., 0., 1., -1., 0.)


def _first_argmax(w, iota, sentinel):
    mx = jnp.max(w, axis=1, keepdims=True)
    return jnp.min(jnp.where(w == mx, iota, sentinel), axis=1)


def _idx_coeff_body(lwa_ref, lwb_ref, maf_ref, mbf_ref, nw_ref, idx_ref, coef_ref):
    rows = lwa_ref.shape[0]
    iota_in = lax.broadcasted_iota(jnp.int32, (rows, IN_DIM), 1)
    wa = jnp.where(maf_ref[...] != 0, lwa_ref[...], -1e30)
    wb = jnp.where(mbf_ref[...] != 0, lwb_ref[...], -1e30)
    ia = _first_argmax(wa, iota_in, IN_DIM)
    ib = _first_argmax(wb, iota_in, IN_DIM)
    zeros_i = jnp.zeros((6, rows), jnp.int32)
    idx_ref[...] = jnp.concatenate([ia[None, :], ib[None, :], zeros_i], axis=0)

    iota_g = lax.broadcasted_iota(jnp.int32, (rows, 16), 1)
    g = _first_argmax(nw_ref[...], iota_g, 16)[None, :]
    c0 = jnp.zeros_like(g, dtype=jnp.float32)
    c1, c2, c3 = c0, c0, c0
    for k in range(16):
        sel = (g == k).astype(jnp.float32)
        c0 = c0 + sel * _C0[k]
        c1 = c1 + sel * _C1[k]
        c2 = c2 + sel * _C2[k]
        c3 = c3 + sel * _C3[k]
    zeros_f = jnp.zeros((4, rows), jnp.float32)
    coef_ref[...] = jnp.concatenate([c0, c1, c2, c3, zeros_f], axis=0)


def _bf16_bits(u):
    lsb = lax.shift_right_logical(u, 16) & 1
    return lax.shift_right_logical(u + 0x7FFF + lsb, 16)


def _pack_body(x0_ref, x1_ref, xt_ref):
    u0 = lax.bitcast_convert_type(x0_ref[...], jnp.int32)
    u1 = lax.bitcast_convert_type(x1_ref[...], jnp.int32)
    packed = lax.shift_left(_bf16_bits(u1), 16) | _bf16_bits(u0)
    xt_ref[...] = jnp.swapaxes(packed, 0, 1)


def _gate_body(a_ref, b_ref, coef_ref, out_ref):
    def unpack(u):
        lo = lax.bitcast_convert_type(lax.shift_left(u, 16), jnp.float32)
        hi = lax.bitcast_convert_type(u & jnp.int32(-65536), jnp.float32)
        return lo, hi

    half = out_ref.shape[0] // 2
    atp = jnp.swapaxes(a_ref[...], 0, 1)
    btp = jnp.swapaxes(b_ref[...], 0, 1)
    a_lo, a_hi = unpack(atp)
    b_lo, b_hi = unpack(btp)
    c0 = coef_ref[0, :][None, :]
    c1 = coef_ref[1, :][None, :]
    c2 = coef_ref[2, :][None, :]
    c3 = coef_ref[3, :][None, :]
    out_ref[pl.ds(0, half), :] = c0 + c1 * a_lo + c2 * b_lo + c3 * (a_lo * b_lo)
    out_ref[pl.ds(half, half), :] = c0 + c1 * a_hi + c2 * b_hi + c3 * (a_hi * b_hi)


def _sc_gather(xTp, idx_flat):
    mesh = plsc.VectorSubcoreMesh(core_axis_name="c", subcore_axis_name="s")
    chunk = 8
    n_chunks = 128 // chunk

    @functools.partial(
        pl.kernel,
        mesh=mesh,
        out_type=jax.ShapeDtypeStruct((2 * OUT_DIM, HALF), jnp.int32),
        scratch_types=[
            pltpu.VMEM((128,), jnp.int32),
            pltpu.VMEM((chunk, HALF), jnp.int32),
            pltpu.VMEM((chunk, HALF), jnp.int32),
            pltpu.VMEM((chunk, HALF), jnp.int32),
            pltpu.SemaphoreType.DMA,
            pltpu.SemaphoreType.DMA,
        ],
    )
    def k(xT_hbm, idx_hbm, out_hbm, idx_v, buf0, buf1, buf2, gsem, wsem):
        wid = lax.axis_index("s") * 2 + lax.axis_index("c")
        base = wid * 128
        pltpu.sync_copy(idx_hbm.at[pl.ds(base, 128)], idx_v)
        bufs = (buf0, buf1, buf2)

        def start_gather(c):
            return pltpu.async_copy(
                xT_hbm.at[idx_v.at[pl.ds(c * chunk, chunk)]], bufs[c % 3], gsem
            )

        g = [None] * n_chunks
        wb = [None] * n_chunks
        g[0] = start_gather(0)
        g[1] = start_gather(1)
        for c in range(n_chunks):
            g[c].wait()
            if c + 2 < n_chunks:
                if c >= 1:
                    wb[c - 1].wait()
                g[c + 2] = start_gather(c + 2)
            wb[c] = pltpu.async_copy(
                bufs[c % 3], out_hbm.at[pl.ds(base + c * chunk, chunk)], wsem
            )
        wb[n_chunks - 3].wait()
        wb[n_chunks - 2].wait()
        wb[n_chunks - 1].wait()

    return k(xTp, idx_flat)


def _stage1(lwa, lwb, maf, mbf, nw, interpret=False):
    blk = 128
    grid = OUT_DIM // blk
    return pl.pallas_call(
        _idx_coeff_body,
        grid=(grid,),
        in_specs=[
            pl.BlockSpec((blk, IN_DIM), lambda g: (g, 0)),
            pl.BlockSpec((blk, IN_DIM), lambda g: (g, 0)),
            pl.BlockSpec((blk, IN_DIM), lambda g: (g, 0)),
            pl.BlockSpec((blk, IN_DIM), lambda g: (g, 0)),
            pl.BlockSpec((blk, 16), lambda g: (g, 0)),
        ],
        out_specs=[
            pl.BlockSpec((8, blk), lambda g: (0, g)),
            pl.BlockSpec((8, blk), lambda g: (0, g)),
        ],
        out_shape=[
            jax.ShapeDtypeStruct((8, OUT_DIM), jnp.int32),
            jax.ShapeDtypeStruct((8, OUT_DIM), jnp.float32),
        ],
        compiler_params=pltpu.CompilerParams(
            dimension_semantics=("parallel",)),
        interpret=interpret,
    )(lwa, lwb, maf, mbf, nw)


def _stage2(x, interpret=False):
    rb, cb = 1024, 1024
    return pl.pallas_call(
        _pack_body,
        grid=(HALF // rb, IN_DIM // cb),
        in_specs=[
            pl.BlockSpec((rb, cb), lambda m, j: (2 * m, j)),
            pl.BlockSpec((rb, cb), lambda m, j: (2 * m + 1, j)),
        ],
        out_specs=pl.BlockSpec((cb, rb), lambda m, j: (j, m)),
        out_shape=jax.ShapeDtypeStruct((IN_DIM, HALF), jnp.int32),
        compiler_params=pltpu.CompilerParams(
            dimension_semantics=("parallel", "parallel")),
        interpret=interpret,
    )(x, x)


def _stage4(gT, coeffs, interpret=False):
    nb, bb = 512, 1024
    return pl.pallas_call(
        _gate_body,
        grid=(OUT_DIM // nb, HALF // bb),
        in_specs=[
            pl.BlockSpec((nb, bb), lambda j, m: (j, m)),
            pl.BlockSpec((nb, bb), lambda j, m: (j + OUT_DIM // nb, m)),
            pl.BlockSpec((8, nb), lambda j, m: (0, j)),
        ],
        out_specs=pl.BlockSpec((2 * bb, nb), lambda j, m: (m, j)),
        out_shape=jax.ShapeDtypeStruct((BATCH, OUT_DIM), jnp.float32),
        compiler_params=pltpu.CompilerParams(
            dimension_semantics=("parallel", "parallel")),
        interpret=interpret,
    )(gT, gT, coeffs)


def kernel(x, neuron_weights, link_weights_a, link_weights_b, link_mask_a, link_mask_b):
    ma8 = link_mask_a.astype(jnp.int8)
    mb8 = link_mask_b.astype(jnp.int8)
    idxmat, coeffs = _stage1(link_weights_a, link_weights_b, ma8,
                             mb8, neuron_weights)
    idx_flat = jnp.concatenate([idxmat[0], idxmat[1]])
    xTp = _stage2(x)
    gT = _sc_gather(xTp, idx_flat)
    return _stage4(gT, coeffs)
